# baseline (speedup 1.0000x reference)
import functools

import jax
import jax.numpy as jnp
from jax import lax
from jax.experimental import pallas as pl
from jax.experimental.pallas import tpu as pltpu
from jax.experimental.pallas import tpu_sc as plsc

NUM_WORKERS = 32
BATCH_SIZE = 16384
PER_WORKER = BATCH_SIZE // NUM_WORKERS
CHUNK = 128
NUM_CHUNKS = PER_WORKER // CHUNK
LANES = 16

_mesh = plsc.VectorSubcoreMesh(core_axis_name="c", subcore_axis_name="s")


@functools.partial(
    pl.kernel,
    mesh=_mesh,
    out_type=jax.ShapeDtypeStruct((BATCH_SIZE,), jnp.float32),
    scratch_types=[
        pltpu.VMEM((1, PER_WORKER), jnp.int32),
        pltpu.VMEM((1, PER_WORKER), jnp.int32),
        pltpu.VMEM((1, PER_WORKER), jnp.float32),
        pltpu.VMEM((1, PER_WORKER), jnp.float32),
        pltpu.VMEM((PER_WORKER,), jnp.float32),
        pltpu.SemaphoreType.DMA,
        pltpu.SemaphoreType.DMA,
        pltpu.SemaphoreType.DMA,
        pltpu.SemaphoreType.DMA,
        pltpu.SemaphoreType.DMA,
        pltpu.SemaphoreType.DMA,
    ],
)
def _irt_sc_kernel(agent_idx_hbm, task_idx_hbm, theta_hbm, beta_hbm,
                   out_hbm, aidx_v, tidx_v, th_v, be_v, o_v, sem_a, sem_t,
                   sem_c0, sem_c1, sem_c2, sem_c3):
    chunk_sems = (sem_c0, sem_c1, sem_c2, sem_c3)
    wid = lax.axis_index("s") * 2 + lax.axis_index("c")
    base = wid * PER_WORKER
    ca = pltpu.async_copy(
        agent_idx_hbm.at[pl.ds(base, PER_WORKER)], aidx_v.at[0], sem_a)
    cb = pltpu.async_copy(
        task_idx_hbm.at[pl.ds(base, PER_WORKER)], tidx_v.at[0], sem_t)
    ca.wait()
    th_copies = []
    for j in range(NUM_CHUNKS):
        sl = pl.ds(j * CHUNK, CHUNK)
        th_copies.append(pltpu.async_copy(
            theta_hbm.at[aidx_v.at[:, sl]], th_v.at[:, sl], chunk_sems[j]))
    cb.wait()
    be_copies = []
    for j in range(NUM_CHUNKS):
        sl = pl.ds(j * CHUNK, CHUNK)
        be_copies.append(pltpu.async_copy(
            beta_hbm.at[tidx_v.at[:, sl]], be_v.at[:, sl], chunk_sems[j]))
    st_copies = []
    for j in range(NUM_CHUNKS):
        th_copies[j].wait()
        be_copies[j].wait()
        for i in range(CHUNK // LANES):
            sl = pl.ds(j * CHUNK + i * LANES, LANES)
            o_v[sl] = th_v[0, sl] - be_v[0, sl]
        csl = pl.ds(j * CHUNK, CHUNK)
        st_copies.append(pltpu.async_copy(
            o_v.at[csl], out_hbm.at[pl.ds(base + j * CHUNK, CHUNK)],
            chunk_sems[j]))
    for c in st_copies:
        c.wait()


def kernel(agent_idx, task_idx, theta, beta):
    return _irt_sc_kernel(
        agent_idx.astype(jnp.int32),
        task_idx.astype(jnp.int32),
        theta.T,
        beta.T,
    )


# speedup vs baseline: 1.0164x; 1.0164x over previous
import functools

import jax
import jax.numpy as jnp
from jax import lax
from jax.experimental import pallas as pl
from jax.experimental.pallas import tpu as pltpu
from jax.experimental.pallas import tpu_sc as plsc

NUM_WORKERS = 16
BATCH_SIZE = 16384
PER_WORKER = BATCH_SIZE // NUM_WORKERS
CHUNK = 256
NUM_CHUNKS = PER_WORKER // CHUNK
LANES = 16

_mesh = plsc.VectorSubcoreMesh(core_axis_name="c", subcore_axis_name="s", num_cores=1)


@functools.partial(
    pl.kernel,
    mesh=_mesh,
    out_type=jax.ShapeDtypeStruct((BATCH_SIZE,), jnp.float32),
    scratch_types=[
        pltpu.VMEM((1, PER_WORKER), jnp.int32),
        pltpu.VMEM((1, PER_WORKER), jnp.int32),
        pltpu.VMEM((1, PER_WORKER), jnp.float32),
        pltpu.VMEM((1, PER_WORKER), jnp.float32),
        pltpu.VMEM((PER_WORKER,), jnp.float32),
        pltpu.SemaphoreType.DMA,
        pltpu.SemaphoreType.DMA,
        pltpu.SemaphoreType.DMA,
        pltpu.SemaphoreType.DMA,
        pltpu.SemaphoreType.DMA,
        pltpu.SemaphoreType.DMA,
    ],
)
def _irt_sc_kernel(agent_idx_hbm, task_idx_hbm, theta_hbm, beta_hbm,
                   out_hbm, aidx_v, tidx_v, th_v, be_v, o_v, sem_a, sem_t,
                   sem_c0, sem_c1, sem_c2, sem_c3):
    chunk_sems = (sem_c0, sem_c1, sem_c2, sem_c3)
    wid = lax.axis_index("s")
    base = wid * PER_WORKER
    ca = pltpu.async_copy(
        agent_idx_hbm.at[pl.ds(base, PER_WORKER)], aidx_v.at[0], sem_a)
    cb = pltpu.async_copy(
        task_idx_hbm.at[pl.ds(base, PER_WORKER)], tidx_v.at[0], sem_t)
    ca.wait()
    th_copies = []
    for j in range(NUM_CHUNKS):
        sl = pl.ds(j * CHUNK, CHUNK)
        th_copies.append(pltpu.async_copy(
            theta_hbm.at[aidx_v.at[:, sl]], th_v.at[:, sl], chunk_sems[j]))
    cb.wait()
    be_copies = []
    for j in range(NUM_CHUNKS):
        sl = pl.ds(j * CHUNK, CHUNK)
        be_copies.append(pltpu.async_copy(
            beta_hbm.at[tidx_v.at[:, sl]], be_v.at[:, sl], chunk_sems[j]))
    st_copies = []
    for j in range(NUM_CHUNKS):
        th_copies[j].wait()
        be_copies[j].wait()
        for i in range(CHUNK // LANES):
            sl = pl.ds(j * CHUNK + i * LANES, LANES)
            o_v[sl] = th_v[0, sl] - be_v[0, sl]
        csl = pl.ds(j * CHUNK, CHUNK)
        st_copies.append(pltpu.async_copy(
            o_v.at[csl], out_hbm.at[pl.ds(base + j * CHUNK, CHUNK)],
            chunk_sems[j]))
    for c in st_copies:
        c.wait()


def kernel(agent_idx, task_idx, theta, beta):
    return _irt_sc_kernel(
        agent_idx.astype(jnp.int32),
        task_idx.astype(jnp.int32),
        theta.T,
        beta.T,
    )


# speedup vs baseline: 1.0196x; 1.0032x over previous
import functools

import jax
import jax.numpy as jnp
from jax import lax
from jax.experimental import pallas as pl
from jax.experimental.pallas import tpu as pltpu
from jax.experimental.pallas import tpu_sc as plsc

NUM_WORKERS = 16
BATCH_SIZE = 16384
PER_WORKER = BATCH_SIZE // NUM_WORKERS
CHUNK = 256
NUM_CHUNKS = PER_WORKER // CHUNK
LANES = 16

_mesh = plsc.VectorSubcoreMesh(core_axis_name="c", subcore_axis_name="s", num_cores=1)


@functools.partial(
    pl.kernel,
    mesh=_mesh,
    out_type=jax.ShapeDtypeStruct((BATCH_SIZE,), jnp.float32),
    scratch_types=[
        pltpu.VMEM((1, PER_WORKER), jnp.int32),
        pltpu.VMEM((1, PER_WORKER), jnp.int32),
        pltpu.VMEM((1, PER_WORKER), jnp.float32),
        pltpu.VMEM((1, PER_WORKER), jnp.float32),
        pltpu.VMEM((PER_WORKER,), jnp.float32),
        pltpu.SemaphoreType.DMA,
        pltpu.SemaphoreType.DMA,
        pltpu.SemaphoreType.DMA,
        pltpu.SemaphoreType.DMA,
        pltpu.SemaphoreType.DMA,
        pltpu.SemaphoreType.DMA,
    ],
)
def _irt_sc_kernel(agent_idx_hbm, task_idx_hbm, theta_hbm, beta_hbm,
                   out_hbm, aidx_v, tidx_v, th_v, be_v, o_v, sem_a, sem_t,
                   sem_c0, sem_c1, sem_c2, sem_c3):
    chunk_sems = (sem_c0, sem_c1, sem_c2, sem_c3)
    wid = lax.axis_index("s")
    base = wid * PER_WORKER
    ca = pltpu.async_copy(
        agent_idx_hbm.at[pl.ds(base, PER_WORKER)], aidx_v.at[0], sem_a)
    cb = pltpu.async_copy(
        task_idx_hbm.at[pl.ds(base, PER_WORKER)], tidx_v.at[0], sem_t)
    ca.wait()
    cb.wait()
    th_copies = []
    be_copies = []
    for j in range(NUM_CHUNKS):
        sl = pl.ds(j * CHUNK, CHUNK)
        th_copies.append(pltpu.async_copy(
            theta_hbm.at[aidx_v.at[:, sl]], th_v.at[:, sl], chunk_sems[j]))
        be_copies.append(pltpu.async_copy(
            beta_hbm.at[tidx_v.at[:, sl]], be_v.at[:, sl], chunk_sems[j]))
    st_copies = []
    for j in range(NUM_CHUNKS):
        th_copies[j].wait()
        be_copies[j].wait()
        for i in range(CHUNK // LANES):
            sl = pl.ds(j * CHUNK + i * LANES, LANES)
            o_v[sl] = th_v[0, sl] - be_v[0, sl]
        csl = pl.ds(j * CHUNK, CHUNK)
        st_copies.append(pltpu.async_copy(
            o_v.at[csl], out_hbm.at[pl.ds(base + j * CHUNK, CHUNK)],
            chunk_sems[j]))
    for c in st_copies:
        c.wait()


def kernel(agent_idx, task_idx, theta, beta):
    return _irt_sc_kernel(
        agent_idx.astype(jnp.int32),
        task_idx.astype(jnp.int32),
        theta.T,
        beta.T,
    )
